# baseline (speedup 1.0000x reference)
import functools

import jax
import jax.numpy as jnp
from jax import lax
from jax.experimental import pallas as pl
from jax.experimental.pallas import tpu as pltpu
from jax.experimental.pallas import tpu_sc as plsc

E = 320000
N = 10000
D = 128
W = D // 2
N_FGROUPS = 4
GROUP_TILES = 8
WPT = W // GROUP_TILES
EPG = E // N_FGROUPS
CHUNK = 1600
PHASE_E = (48000, 32000)
PHASE_C = (0, PHASE_E[0] // CHUNK)
PHASE_N = (PHASE_E[0] // CHUNK, PHASE_E[1] // CHUNK)
RED_SUB = 2000


def _sc_body(ht_hbm, ei_hbm, out_hbm,
             parts_sp, table_v, idx_s0, idx_s1, idx_d0, idx_d1,
             part0, part1, red_v, out_all,
             sem_i0, sem_i1, sem_p0, sem_p1):
    cid = lax.axis_index("c")
    sid = lax.axis_index("s")
    g = sid // GROUP_TILES
    r = sid % GROUP_TILES
    p = cid * 2 + g
    ebase = p * EPG
    idx_s = (idx_s0, idx_s1)
    idx_d = (idx_d0, idx_d1)
    part = (part0, part1)
    sem_i = (sem_i0, sem_i1)
    sem_p = (sem_p0, sem_p1)

    pltpu.sync_copy(
        ht_hbm.at[pl.ds(0, N), pl.ds(r * WPT, WPT)], table_v)

    def issue_idx(gi, b):
        off = pl.multiple_of(ebase + gi * CHUNK, 8)
        pltpu.async_copy(ei_hbm.at[0, pl.ds(off, CHUNK)], idx_s[b], sem_i[b])
        pltpu.async_copy(ei_hbm.at[1, pl.ds(off, CHUNK)], idx_d[b], sem_i[b])

    def wait_idx(gi, b):
        off = pl.multiple_of(ebase + gi * CHUNK, 8)
        pltpu.make_async_copy(
            ei_hbm.at[0, pl.ds(off, CHUNK)], idx_s[b], sem_i[b]).wait()
        pltpu.make_async_copy(
            ei_hbm.at[1, pl.ds(off, CHUNK)], idx_d[b], sem_i[b]).wait()

    def part_dst(ci):
        return parts_sp.at[
            g, r, pl.ds(pl.multiple_of(ci * (CHUNK // 2), 8), CHUNK // 2)]

    lane2 = lax.iota(jnp.int32, 16) * 2

    def dot16(ks, kd):
        acc = jnp.zeros((32,), jnp.bfloat16)
        for w in range(WPT):
            wv = jnp.full((16,), w, jnp.int32)
            sw = plsc.load_gather(table_v, [ks, wv])
            dw = plsc.load_gather(table_v, [kd, wv])
            sb = plsc.bitcast(sw, jnp.bfloat16)
            db = plsc.bitcast(dw, jnp.bfloat16)
            acc = acc + sb * db
        a1, a2 = plsc.unpack(acc, format=plsc.PackFormat.INTERLEAVED)
        return a1 + a2

    def compute(ci, b):
        isv, idv, pv = idx_s[b], idx_d[b], part[b]

        def group_body(j, carry):
            base32 = j * 32
            ks_a = plsc.load_gather(isv, [lane2 + base32])
            ks_b = plsc.load_gather(isv, [lane2 + (base32 + 1)])
            kd_a = plsc.load_gather(idv, [lane2 + base32])
            kd_b = plsc.load_gather(idv, [lane2 + (base32 + 1)])
            acc_a = dot16(ks_a, kd_a)
            acc_b = dot16(ks_b, kd_b)
            packed = plsc.bitcast(
                plsc.pack(acc_a, acc_b, format=plsc.PackFormat.INTERLEAVED),
                jnp.int32)
            pv[pl.ds(pl.multiple_of(j * 16, 16), 16)] = packed
            return carry

        lax.fori_loop(0, CHUNK // 32, group_body, 0)
        pltpu.async_copy(pv, part_dst(ci), sem_p[b])

    def wait_part(ci, b):
        pltpu.make_async_copy(part[b], part_dst(ci), sem_p[b]).wait()

    for ph in range(2):
        pc0, nc = PHASE_C[ph], PHASE_N[ph]
        ept_ph = PHASE_E[ph] // GROUP_TILES
        n_red = ept_ph // RED_SUB

        issue_idx(pc0, 0)

        def pair_body(i2, carry):
            for b in range(2):
                ci = i2 * 2 + b
                issue_idx(pc0 + ci + 1, 1 - b)
                wait_idx(pc0 + ci, b)

                @pl.when(ci >= 2)
                def _drain():
                    wait_part(ci - 2, b)

                compute(ci, b)
            return carry

        lax.fori_loop(0, (nc - 2) // 2, pair_body, 0)
        c0, c1 = nc - 2, nc - 1
        issue_idx(pc0 + c1, c1 % 2)
        wait_idx(pc0 + c0, c0 % 2)
        wait_part(c0 - 2, c0 % 2)
        compute(c0, c0 % 2)
        wait_idx(pc0 + c1, c1 % 2)
        wait_part(c1 - 2, c1 % 2)
        compute(c1, c1 % 2)
        wait_part(c0, c0 % 2)
        wait_part(c1, c1 % 2)

        plsc.subcore_barrier()

        for sub in range(n_red):
            soff = pl.multiple_of((r * ept_ph + sub * RED_SUB) // 2, 8)
            pltpu.sync_copy(
                parts_sp.at[g, :, pl.ds(soff, RED_SUB // 2)], red_v)

            def red_body(j, carry):
                base16 = jnp.minimum(j * 16, RED_SUB // 2 - 16)
                acc_a = jnp.zeros((16,), jnp.float32)
                acc_b = jnp.zeros((16,), jnp.float32)
                for t in range(GROUP_TILES):
                    pb = plsc.bitcast(
                        red_v[t, pl.ds(base16, 16)], jnp.bfloat16)
                    a, b2 = plsc.unpack(
                        pb, format=plsc.PackFormat.INTERLEAVED)
                    acc_a = acc_a + a
                    acc_b = acc_b + b2
                obase = sub * RED_SUB + base16 * 2
                plsc.store_scatter(out_all, [lane2 + obase], acc_a)
                plsc.store_scatter(out_all, [lane2 + (obase + 1)], acc_b)
                return carry

            lax.fori_loop(0, (RED_SUB // 2 + 15) // 16, red_body, 0)

        obase = pl.multiple_of(p * EPG + pc0 * CHUNK + r * ept_ph, 8)
        pltpu.sync_copy(
            out_all.at[pl.ds(0, ept_ph)], out_hbm.at[pl.ds(obase, ept_ph)])

        plsc.subcore_barrier()


@jax.jit
def _sc_call(ht, ei):
    mesh = plsc.VectorSubcoreMesh(core_axis_name="c", subcore_axis_name="s")
    fn = pl.kernel(
        _sc_body,
        out_type=jax.ShapeDtypeStruct((E,), jnp.float32),
        mesh=mesh,
        compiler_params=pltpu.CompilerParams(
            needs_layout_passes=False, use_tc_tiling_on_sc=False),
        scratch_types=[
            pltpu.VMEM_SHARED((2, GROUP_TILES, PHASE_E[0] // 2), jnp.int32),
            pltpu.VMEM((N, WPT), jnp.int32),
            pltpu.VMEM((CHUNK,), jnp.int32),
            pltpu.VMEM((CHUNK,), jnp.int32),
            pltpu.VMEM((CHUNK,), jnp.int32),
            pltpu.VMEM((CHUNK,), jnp.int32),
            pltpu.VMEM((CHUNK // 2,), jnp.int32),
            pltpu.VMEM((CHUNK // 2,), jnp.int32),
            pltpu.VMEM((GROUP_TILES, RED_SUB // 2), jnp.int32),
            pltpu.VMEM((PHASE_E[0] // GROUP_TILES,), jnp.float32),
            pltpu.SemaphoreType.DMA,
            pltpu.SemaphoreType.DMA,
            pltpu.SemaphoreType.DMA,
            pltpu.SemaphoreType.DMA,
        ],
    )
    return fn(ht, ei)


def kernel(h, edge_index):
    h_packed = jax.lax.bitcast_convert_type(
        h.astype(jnp.bfloat16).reshape(N, W, 2), jnp.int32)
    out = _sc_call(h_packed, edge_index.astype(jnp.int32))
    return out.reshape(E, 1)


# speedup vs baseline: 1.4174x; 1.4174x over previous
import functools

import jax
import jax.numpy as jnp
from jax import lax
from jax.experimental import pallas as pl
from jax.experimental.pallas import tpu as pltpu
from jax.experimental.pallas import tpu_sc as plsc

E = 320000
N = 10000
D = 128
W = D // 2
N_FGROUPS = 4
GROUP_TILES = 8
WPT = W // GROUP_TILES
EPG = E // N_FGROUPS
CHUNK = 1600
PHASE_E = (48000, 32000)
PHASE_C = (0, PHASE_E[0] // CHUNK)
PHASE_N = (PHASE_E[0] // CHUNK, PHASE_E[1] // CHUNK)
RED_SUB = 2000


def _sc_body(ht_hbm, ei_hbm, out_hbm,
             parts_sp, table_v, idx_s0, idx_s1, idx_d0, idx_d1,
             part0, part1, red_v, out_all,
             sem_i0, sem_i1, sem_p0, sem_p1):
    cid = lax.axis_index("c")
    sid = lax.axis_index("s")
    g = sid // GROUP_TILES
    r = sid % GROUP_TILES
    p = cid * 2 + g
    ebase = p * EPG
    idx_s = (idx_s0, idx_s1)
    idx_d = (idx_d0, idx_d1)
    part = (part0, part1)
    sem_i = (sem_i0, sem_i1)
    sem_p = (sem_p0, sem_p1)

    pltpu.sync_copy(
        ht_hbm.at[pl.ds(pl.multiple_of(r * WPT * N, 8), WPT * N)], table_v)

    def issue_idx(gi, b):
        off = pl.multiple_of(ebase + gi * CHUNK, 8)
        pltpu.async_copy(ei_hbm.at[0, pl.ds(off, CHUNK)], idx_s[b], sem_i[b])
        pltpu.async_copy(ei_hbm.at[1, pl.ds(off, CHUNK)], idx_d[b], sem_i[b])

    def wait_idx(gi, b):
        off = pl.multiple_of(ebase + gi * CHUNK, 8)
        pltpu.make_async_copy(
            ei_hbm.at[0, pl.ds(off, CHUNK)], idx_s[b], sem_i[b]).wait()
        pltpu.make_async_copy(
            ei_hbm.at[1, pl.ds(off, CHUNK)], idx_d[b], sem_i[b]).wait()

    def part_dst(ci):
        return parts_sp.at[
            g, r, pl.ds(pl.multiple_of(ci * (CHUNK // 2), 8), CHUNK // 2)]

    lane2 = lax.iota(jnp.int32, 16) * 2

    def dot16(ks, kd):
        acc = jnp.zeros((32,), jnp.bfloat16)
        for w in range(WPT):
            sw = plsc.load_gather(table_v, [ks + (w * N)])
            dw = plsc.load_gather(table_v, [kd + (w * N)])
            sb = plsc.bitcast(sw, jnp.bfloat16)
            db = plsc.bitcast(dw, jnp.bfloat16)
            acc = acc + sb * db
        a1, a2 = plsc.unpack(acc, format=plsc.PackFormat.INTERLEAVED)
        return a1 + a2

    def compute(ci, b):
        isv, idv, pv = idx_s[b], idx_d[b], part[b]

        def group_body(j, carry):
            base32 = j * 32
            ks_a = plsc.load_gather(isv, [lane2 + base32])
            ks_b = plsc.load_gather(isv, [lane2 + (base32 + 1)])
            kd_a = plsc.load_gather(idv, [lane2 + base32])
            kd_b = plsc.load_gather(idv, [lane2 + (base32 + 1)])
            acc_a = dot16(ks_a, kd_a)
            acc_b = dot16(ks_b, kd_b)
            packed = plsc.bitcast(
                plsc.pack(acc_a, acc_b, format=plsc.PackFormat.INTERLEAVED),
                jnp.int32)
            pv[pl.ds(pl.multiple_of(j * 16, 16), 16)] = packed
            return carry

        lax.fori_loop(0, CHUNK // 32, group_body, 0)
        pltpu.async_copy(pv, part_dst(ci), sem_p[b])

    def wait_part(ci, b):
        pltpu.make_async_copy(part[b], part_dst(ci), sem_p[b]).wait()

    for ph in range(2):
        pc0, nc = PHASE_C[ph], PHASE_N[ph]
        ept_ph = PHASE_E[ph] // GROUP_TILES
        n_red = ept_ph // RED_SUB

        issue_idx(pc0, 0)

        def pair_body(i2, carry):
            for b in range(2):
                ci = i2 * 2 + b
                issue_idx(pc0 + ci + 1, 1 - b)
                wait_idx(pc0 + ci, b)

                @pl.when(ci >= 2)
                def _drain():
                    wait_part(ci - 2, b)

                compute(ci, b)
            return carry

        lax.fori_loop(0, (nc - 2) // 2, pair_body, 0)
        c0, c1 = nc - 2, nc - 1
        issue_idx(pc0 + c1, c1 % 2)
        wait_idx(pc0 + c0, c0 % 2)
        wait_part(c0 - 2, c0 % 2)
        compute(c0, c0 % 2)
        wait_idx(pc0 + c1, c1 % 2)
        wait_part(c1 - 2, c1 % 2)
        compute(c1, c1 % 2)
        wait_part(c0, c0 % 2)
        wait_part(c1, c1 % 2)

        plsc.subcore_barrier()

        for sub in range(n_red):
            soff = pl.multiple_of((r * ept_ph + sub * RED_SUB) // 2, 8)
            pltpu.sync_copy(
                parts_sp.at[g, :, pl.ds(soff, RED_SUB // 2)], red_v)

            def red_body(j, carry):
                base16 = jnp.minimum(j * 16, RED_SUB // 2 - 16)
                acc_a = jnp.zeros((16,), jnp.float32)
                acc_b = jnp.zeros((16,), jnp.float32)
                for t in range(GROUP_TILES):
                    pb = plsc.bitcast(
                        red_v[t, pl.ds(base16, 16)], jnp.bfloat16)
                    a, b2 = plsc.unpack(
                        pb, format=plsc.PackFormat.INTERLEAVED)
                    acc_a = acc_a + a
                    acc_b = acc_b + b2
                obase = sub * RED_SUB + base16 * 2
                plsc.store_scatter(out_all, [lane2 + obase], acc_a)
                plsc.store_scatter(out_all, [lane2 + (obase + 1)], acc_b)
                return carry

            lax.fori_loop(0, (RED_SUB // 2 + 15) // 16, red_body, 0)

        obase = pl.multiple_of(p * EPG + pc0 * CHUNK + r * ept_ph, 8)
        pltpu.sync_copy(
            out_all.at[pl.ds(0, ept_ph)], out_hbm.at[pl.ds(obase, ept_ph)])

        plsc.subcore_barrier()


@jax.jit
def _sc_call(ht, ei):
    mesh = plsc.VectorSubcoreMesh(core_axis_name="c", subcore_axis_name="s")
    fn = pl.kernel(
        _sc_body,
        out_type=jax.ShapeDtypeStruct((E,), jnp.float32),
        mesh=mesh,
        compiler_params=pltpu.CompilerParams(
            needs_layout_passes=False, use_tc_tiling_on_sc=False),
        scratch_types=[
            pltpu.VMEM_SHARED((2, GROUP_TILES, PHASE_E[0] // 2), jnp.int32),
            pltpu.VMEM((WPT * N,), jnp.int32),
            pltpu.VMEM((CHUNK,), jnp.int32),
            pltpu.VMEM((CHUNK,), jnp.int32),
            pltpu.VMEM((CHUNK,), jnp.int32),
            pltpu.VMEM((CHUNK,), jnp.int32),
            pltpu.VMEM((CHUNK // 2,), jnp.int32),
            pltpu.VMEM((CHUNK // 2,), jnp.int32),
            pltpu.VMEM((GROUP_TILES, RED_SUB // 2), jnp.int32),
            pltpu.VMEM((PHASE_E[0] // GROUP_TILES,), jnp.float32),
            pltpu.SemaphoreType.DMA,
            pltpu.SemaphoreType.DMA,
            pltpu.SemaphoreType.DMA,
            pltpu.SemaphoreType.DMA,
        ],
    )
    return fn(ht, ei)


def kernel(h, edge_index):
    h_packed = jax.lax.bitcast_convert_type(
        h.astype(jnp.bfloat16).reshape(N, W, 2), jnp.int32)
    ht = h_packed.T.reshape(W * N)
    out = _sc_call(ht, edge_index.astype(jnp.int32))
    return out.reshape(E, 1)
